# baseline (speedup 1.0000x reference)
import functools

import jax
import jax.numpy as jnp
from jax import lax
from jax.experimental import pallas as pl
from jax.experimental.pallas import tpu as pltpu
from jax.experimental.pallas import tpu_sc as plsc

_NUM_CORES = 2
_NUM_SUBCORES = 16
_NW = _NUM_CORES * _NUM_SUBCORES

_BL = 128
_NBUF = 4
_TPAD = 132


@functools.lru_cache(maxsize=None)
def _make_sc_relayout(v: int, d: int):
  assert d == 64
  jt = (v + 127) // 128
  vp = jt * 128
  nbuf = 4
  kt = ((jt + _NW - 1) // _NW + nbuf - 1) // nbuf * nbuf
  g_total = kt // nbuf - 2
  tp = 136

  mesh = plsc.VectorSubcoreMesh(core_axis_name="c", subcore_axis_name="s")

  @functools.partial(
      pl.kernel,
      mesh=mesh,
      out_type=jax.ShapeDtypeStruct((vp // 2, 128), jnp.float32),
      compiler_params=pltpu.CompilerParams(needs_layout_passes=False),
      scratch_types=(
          [pltpu.VMEM((d, 128), jnp.float32) for _ in range(nbuf)]
          + [pltpu.VMEM((d, tp), jnp.float32) for _ in range(nbuf)]
          + [pltpu.SemaphoreType.DMA for _ in range(2 * nbuf)]
      ),
  )
  def sc_relayout(tab_t, out_hbm, *rest):
    ibufs = rest[:nbuf]
    tbufs = rest[nbuf:2 * nbuf]
    isems = rest[2 * nbuf:3 * nbuf]
    wsems = rest[3 * nbuf:]
    wid = lax.axis_index("s") * _NUM_CORES + lax.axis_index("c")

    def jcol(k):
      return wid + _NW * k

    def start_in(b, k):
      @pl.when(jcol(k) < jt)
      def _():
        pltpu.async_copy(
            tab_t.at[:, pl.ds(128 * jcol(k), 128)], ibufs[b], isems[b])

    def wait_in(b, k):
      @pl.when(jcol(k) < jt)
      def _():
        pltpu.make_async_copy(
            tab_t.at[:, pl.ds(128 * jcol(k), 128)], ibufs[b], isems[b]).wait()

    def transpose(b, k):
      ibuf, tbuf = ibufs[b], tbufs[b]

      @pl.when(jcol(k) < jt)
      def _():
        def tdiag(r0, carry):
          lv = lax.iota(jnp.int32, 16)
          rv = lax.rem(lv + r0, 128)
          wv = rv // 2
          lpar = (rv % 2) * 64
          for cb in range(d // 16):
            cv = lv + 16 * cb
            val = plsc.load_gather(ibuf, [cv, rv])
            plsc.store_scatter(tbuf, [wv, lpar + cv], val)
          return carry

        lax.fori_loop(0, 128, tdiag, 0)

    def start_write(b, k):
      @pl.when(jcol(k) < jt)
      def _():
        pltpu.async_copy(
            tbufs[b].at[:, :128], out_hbm.at[pl.ds(64 * jcol(k), 64)],
            wsems[b])

    def wait_write(b, k):
      @pl.when(jcol(k) < jt)
      def _():
        pltpu.make_async_copy(
            tbufs[b].at[:, :128], out_hbm.at[pl.ds(64 * jcol(k), 64)],
            wsems[b]).wait()

    for b in range(nbuf):
      start_in(b, b)
    for t in range(nbuf):
      wait_in(t, t)
      transpose(t, t)
      start_write(t, t)
      start_in(t, t + nbuf)

    def body(g, carry):
      k0 = nbuf + g * nbuf
      for b in range(nbuf):
        k = k0 + b
        wait_in(b, k)
        wait_write(b, k - nbuf)
        transpose(b, k)
        start_write(b, k)
        start_in(b, k + nbuf)
      return carry

    lax.fori_loop(0, g_total, body, 0)

    for b in range(nbuf):
      k = kt - nbuf + b
      wait_in(b, k)
      wait_write(b, k - nbuf)
      transpose(b, k)
      start_write(b, k)
    for b in range(nbuf):
      wait_write(b, kt - nbuf + b)

  return sc_relayout


@functools.lru_cache(maxsize=None)
def _make_sc_gather(batch: int, seq: int, d: int):
  assert batch == _NW * _BL and seq % 8 == 0 and d == 64
  si_n, su_n, ci_n = seq // 8, 8, d // 8
  nch = seq
  nbuf = _NBUF
  assert nch % nbuf == 0 and nch >= 2 * nbuf
  g_total = nch // nbuf - 2

  mesh = plsc.VectorSubcoreMesh(core_axis_name="c", subcore_axis_name="s")

  @functools.partial(
      pl.kernel,
      mesh=mesh,
      out_type=jax.ShapeDtypeStruct((seq, ci_n, _NW, su_n, _BL), jnp.float32),
      compiler_params=pltpu.CompilerParams(
          use_tc_tiling_on_sc=False, needs_layout_passes=False),
      scratch_types=(
          [pltpu.VMEM((si_n, su_n, _BL), jnp.int32)]
          + [pltpu.VMEM((_BL, d), jnp.float32) for _ in range(nbuf)]
          + [pltpu.VMEM((ci_n * su_n, _TPAD), jnp.float32) for _ in range(nbuf)]
          + [pltpu.SemaphoreType.DMA for _ in range(2 * nbuf)]
      ),
  )
  def sc_gather(idx_hbm, table_hbm, out_hbm, idx_v, *rest):
    rbufs = rest[:nbuf]
    tbufs = rest[nbuf:2 * nbuf]
    gsems = rest[2 * nbuf:3 * nbuf]
    wsems = rest[3 * nbuf:]
    wid = lax.axis_index("s") * _NUM_CORES + lax.axis_index("c")

    pltpu.sync_copy(idx_hbm.at[:, wid], idx_v)

    def idx_ref(s):
      return idx_v.at[s // 8, s % 8]

    def start_gather(b, s):
      pltpu.async_copy(table_hbm.at[idx_ref(s)], rbufs[b], gsems[b])

    def wait_gather(b, s):
      pltpu.make_async_copy(
          table_hbm.at[idx_ref(s)], rbufs[b], gsems[b]).wait()

    def transpose(b):
      rbuf, tbuf = rbufs[b], tbufs[b]

      def trow(r0, carry):
        lv = lax.iota(jnp.int32, 16)
        rv = lax.rem(lv + r0, _BL)
        for q in range(d // 16):
          cv = lv + 16 * q
          v = plsc.load_gather(rbuf, [rv, cv])
          plsc.store_scatter(tbuf, [cv, rv], v)
        return carry

      lax.fori_loop(0, _BL, trow, 0)

    def start_write(b, s):
      for ci in range(ci_n):
        pltpu.async_copy(
            tbufs[b].at[pl.ds(8 * ci, 8), :_BL], out_hbm.at[s, ci, wid],
            wsems[b])

    def wait_write(b, s):
      for ci in range(ci_n):
        pltpu.make_async_copy(
            tbufs[b].at[pl.ds(8 * ci, 8), :_BL], out_hbm.at[s, ci, wid],
            wsems[b]).wait()

    for b in range(nbuf):
      start_gather(b, b)
    for t in range(nbuf):
      wait_gather(t, t)
      transpose(t)
      start_write(t, t)
      start_gather(t, t + nbuf)

    def body(g, carry):
      t0 = nbuf + g * nbuf
      for b in range(nbuf):
        t = t0 + b
        wait_gather(b, t)
        wait_write(b, t - nbuf)
        transpose(b)
        start_write(b, t)
        start_gather(b, t + nbuf)
      return carry

    lax.fori_loop(0, g_total, body, 0)

    for b in range(nbuf):
      t = nch - nbuf + b
      wait_gather(b, t)
      wait_write(b, t - nbuf)
      transpose(b)
      start_write(b, t)
    for b in range(nbuf):
      wait_write(b, nch - nbuf + b)

  return sc_gather


def kernel(indices, table):
  b, s = indices.shape
  v, d = table.shape
  idx_p = (indices.astype(jnp.int32).T
           .reshape(s // 8, 8, b // 128, 128).transpose(0, 2, 1, 3))
  tbl_rm = _make_sc_relayout(v, d)(table.T).reshape(-1, d)
  out5 = _make_sc_gather(b, s, d)(idx_p, tbl_rm)
  return out5.transpose(2, 4, 0, 1, 3).reshape(b, s, d)


# speedup vs baseline: 2.9160x; 2.9160x over previous
import functools

import jax
import jax.numpy as jnp
from jax import lax
from jax.experimental import pallas as pl
from jax.experimental.pallas import tpu as pltpu
from jax.experimental.pallas import tpu_sc as plsc

_NUM_CORES = 2
_NUM_SUBCORES = 16
_NW = _NUM_CORES * _NUM_SUBCORES

_BL = 128
_NBUF = 4
_TPAD = 133


@functools.lru_cache(maxsize=None)
def _make_sc_relayout(v: int, d: int):
  assert d == 64
  jt = (v + 127) // 128
  vp = jt * 128
  nbuf = 4
  kt = ((jt + _NW - 1) // _NW + nbuf - 1) // nbuf * nbuf
  g_total = kt // nbuf - 2
  tp = 136

  mesh = plsc.VectorSubcoreMesh(core_axis_name="c", subcore_axis_name="s")

  @functools.partial(
      pl.kernel,
      mesh=mesh,
      out_type=jax.ShapeDtypeStruct((vp // 2, 128), jnp.float32),
      compiler_params=pltpu.CompilerParams(needs_layout_passes=False),
      scratch_types=(
          [pltpu.VMEM((d, 128), jnp.float32) for _ in range(nbuf)]
          + [pltpu.VMEM((d, tp), jnp.float32) for _ in range(nbuf)]
          + [pltpu.SemaphoreType.DMA for _ in range(2 * nbuf)]
      ),
  )
  def sc_relayout(tab_t, out_hbm, *rest):
    ibufs = rest[:nbuf]
    tbufs = rest[nbuf:2 * nbuf]
    isems = rest[2 * nbuf:3 * nbuf]
    wsems = rest[3 * nbuf:]
    wid = lax.axis_index("s") * _NUM_CORES + lax.axis_index("c")

    def jcol(k):
      return wid + _NW * k

    def start_in(b, k):
      @pl.when(jcol(k) < jt)
      def _():
        pltpu.async_copy(
            tab_t.at[:, pl.ds(128 * jcol(k), 128)], ibufs[b], isems[b])

    def wait_in(b, k):
      @pl.when(jcol(k) < jt)
      def _():
        pltpu.make_async_copy(
            tab_t.at[:, pl.ds(128 * jcol(k), 128)], ibufs[b], isems[b]).wait()

    def transpose(b, k):
      ibuf, tbuf = ibufs[b], tbufs[b]

      @pl.when(jcol(k) < jt)
      def _():
        @plsc.parallel_loop(0, 128, unroll=4)
        def _tdiag(r0):
          lv = lax.iota(jnp.int32, 16)
          rv = lax.rem(lv + r0, 128)
          wv = rv // 2
          lpar = (rv % 2) * 64
          for cb in range(d // 16):
            cv = lv + 16 * cb
            val = plsc.load_gather(ibuf, [cv, rv])
            plsc.store_scatter(tbuf, [wv, lpar + cv], val)

    def start_write(b, k):
      @pl.when(jcol(k) < jt)
      def _():
        pltpu.async_copy(
            tbufs[b].at[:, :128], out_hbm.at[pl.ds(64 * jcol(k), 64)],
            wsems[b])

    def wait_write(b, k):
      @pl.when(jcol(k) < jt)
      def _():
        pltpu.make_async_copy(
            tbufs[b].at[:, :128], out_hbm.at[pl.ds(64 * jcol(k), 64)],
            wsems[b]).wait()

    for b in range(nbuf):
      start_in(b, b)
    for t in range(nbuf):
      wait_in(t, t)
      transpose(t, t)
      start_write(t, t)
      start_in(t, t + nbuf)

    def body(g, carry):
      k0 = nbuf + g * nbuf
      for b in range(nbuf):
        k = k0 + b
        wait_in(b, k)
        wait_write(b, k - nbuf)
        transpose(b, k)
        start_write(b, k)
        start_in(b, k + nbuf)
      return carry

    lax.fori_loop(0, g_total, body, 0)

    for b in range(nbuf):
      k = kt - nbuf + b
      wait_in(b, k)
      wait_write(b, k - nbuf)
      transpose(b, k)
      start_write(b, k)
    for b in range(nbuf):
      wait_write(b, kt - nbuf + b)

  return sc_relayout


@functools.lru_cache(maxsize=None)
def _make_sc_gather(batch: int, seq: int, d: int):
  assert batch == _NW * _BL and seq % 8 == 0 and d == 64
  si_n, su_n, ci_n = seq // 8, 8, d // 8
  nch = seq
  nbuf = _NBUF
  assert nch % nbuf == 0 and nch >= 2 * nbuf
  g_total = nch // nbuf - 2

  mesh = plsc.VectorSubcoreMesh(core_axis_name="c", subcore_axis_name="s")

  @functools.partial(
      pl.kernel,
      mesh=mesh,
      out_type=jax.ShapeDtypeStruct((seq, ci_n, _NW, su_n, _BL), jnp.float32),
      compiler_params=pltpu.CompilerParams(
          use_tc_tiling_on_sc=False, needs_layout_passes=False),
      scratch_types=(
          [pltpu.VMEM((si_n, su_n, _BL), jnp.int32)]
          + [pltpu.VMEM((_BL, d), jnp.float32) for _ in range(nbuf)]
          + [pltpu.VMEM((ci_n * su_n, _TPAD), jnp.float32) for _ in range(nbuf)]
          + [pltpu.SemaphoreType.DMA for _ in range(2 * nbuf)]
      ),
  )
  def sc_gather(idx_hbm, table_hbm, out_hbm, idx_v, *rest):
    rbufs = rest[:nbuf]
    tbufs = rest[nbuf:2 * nbuf]
    gsems = rest[2 * nbuf:3 * nbuf]
    wsems = rest[3 * nbuf:]
    wid = lax.axis_index("s") * _NUM_CORES + lax.axis_index("c")

    pltpu.sync_copy(idx_hbm.at[:, wid], idx_v)

    def idx_ref(s):
      return idx_v.at[s // 8, s % 8]

    def start_gather(b, s):
      pltpu.async_copy(table_hbm.at[idx_ref(s)], rbufs[b], gsems[b])

    def wait_gather(b, s):
      pltpu.make_async_copy(
          table_hbm.at[idx_ref(s)], rbufs[b], gsems[b]).wait()

    def transpose(b):
      rbuf, tbuf = rbufs[b], tbufs[b]

      @plsc.parallel_loop(0, _BL, unroll=4)
      def _trow(r):
        lv = lax.iota(jnp.int32, 16)
        rs = lv * 0 + r
        for q in range(d // 16):
          v = rbuf[r, pl.ds(16 * q, 16)]
          plsc.store_scatter(tbuf, [lv + 16 * q, rs], v)

    def start_write(b, s):
      for ci in range(ci_n):
        pltpu.async_copy(
            tbufs[b].at[pl.ds(8 * ci, 8), :_BL], out_hbm.at[s, ci, wid],
            wsems[b])

    def wait_write(b, s):
      for ci in range(ci_n):
        pltpu.make_async_copy(
            tbufs[b].at[pl.ds(8 * ci, 8), :_BL], out_hbm.at[s, ci, wid],
            wsems[b]).wait()

    for b in range(nbuf):
      start_gather(b, b)
    for t in range(nbuf):
      wait_gather(t, t)
      transpose(t)
      start_write(t, t)
      start_gather(t, t + nbuf)

    def body(g, carry):
      t0 = nbuf + g * nbuf
      for b in range(nbuf):
        t = t0 + b
        wait_gather(b, t)
        wait_write(b, t - nbuf)
        transpose(b)
        start_write(b, t)
        start_gather(b, t + nbuf)
      return carry

    lax.fori_loop(0, g_total, body, 0)

    for b in range(nbuf):
      t = nch - nbuf + b
      wait_gather(b, t)
      wait_write(b, t - nbuf)
      transpose(b)
      start_write(b, t)
    for b in range(nbuf):
      wait_write(b, nch - nbuf + b)

  return sc_gather


def kernel(indices, table):
  b, s = indices.shape
  v, d = table.shape
  idx_p = (indices.astype(jnp.int32).T
           .reshape(s // 8, 8, b // 128, 128).transpose(0, 2, 1, 3))
  tbl_rm = _make_sc_relayout(v, d)(table.T).reshape(-1, d)
  out5 = _make_sc_gather(b, s, d)(idx_p, tbl_rm)
  return out5.transpose(2, 4, 0, 1, 3).reshape(b, s, d)
